# baseline (speedup 1.0000x reference)
import functools

import jax
import jax.numpy as jnp
from jax import lax
from jax.experimental import pallas as pl
from jax.experimental.pallas import tpu as pltpu
from jax.experimental.pallas import tpu_sc as plsc

B = 16384
L = 50
E = 32
V = 1_000_000

BLKW = 262144
RB = BLKW // 128
SB = BLKW // E
NBLK = (V * E + BLKW - 1) // BLKW
SPAD = NBLK * SB


def _dot_body(wm_ref, x_ref, o_ref):
    i = pl.program_id(0)
    x2 = x_ref[...].reshape(RB, 128)
    prod = lax.dot_general(wm_ref[...], x2,
                           (((1,), (1,)), ((), ())),
                           preferred_element_type=jnp.float32)
    for t in range(4):
        o_ref[pl.ds(pl.multiple_of(i * SB + t * RB, 2048), RB)] = (
            prod[t:t + 1, :].reshape(RB))


def _table_dot_w(emb_table, w_row):
    k = jnp.arange(128)
    wm4 = jnp.where((k[None, :] // E) == jnp.arange(4)[:, None],
                    jnp.tile(w_row, 4)[None, :], 0.0).astype(jnp.float32)
    flat = emb_table.reshape(V * E)
    return pl.pallas_call(
        _dot_body,
        grid=(NBLK,),
        in_specs=[
            pl.BlockSpec((4, 128), lambda i: (0, 0)),
            pl.BlockSpec((BLKW,), lambda i: (i,)),
        ],
        out_specs=pl.BlockSpec((SPAD,), lambda i: (0,)),
        out_shape=jax.ShapeDtypeStruct((SPAD,), jnp.float32),
    )(wm4, flat)


NC, NS = 2, 16
NW = NC * NS
NB_PER_TILE = B // NW
NIDX = NB_PER_TILE * L
CHUNK = 128
NCHUNK = NIDX // CHUNK
GROUP = 8
LPAD = 64

_mesh = plsc.VectorSubcoreMesh(core_axis_name="c", subcore_axis_name="s")


@functools.partial(
    pl.kernel,
    mesh=_mesh,
    out_type=jax.ShapeDtypeStruct((NW, LPAD), jnp.float32),
    scratch_types=[
        pltpu.VMEM((NIDX,), jnp.int32),
        pltpu.VMEM((NIDX + LPAD,), jnp.float32),
        pltpu.VMEM((LPAD,), jnp.float32),
        pltpu.SemaphoreType.DMA,
    ],
)
def _sc_segment_sum(inp_hbm, s_hbm, out_hbm, idx_v, vals_v, acc_v, sem):
    cid = lax.axis_index("c")
    sid = lax.axis_index("s")
    wid = sid * NC + cid
    base = wid * NIDX

    pltpu.sync_copy(inp_hbm.at[pl.ds(base, NIDX)], idx_v)

    def remap(q, carry):
        r = idx_v[pl.ds(q * 16, 16)]
        rem = lax.bitwise_and(r, 8191)
        sidx = (lax.bitwise_and(r, jnp.int32(~8191))
                + lax.shift_left(lax.bitwise_and(r, 3), 11)
                + lax.shift_right_logical(rem, 2))
        idx_v[pl.ds(q * 16, 16)] = sidx
        return carry

    lax.fori_loop(0, NIDX // 16, remap, 0)

    def gather_group(g, carry):
        for j in range(GROUP):
            off = (g * GROUP + j) * CHUNK
            pltpu.async_copy(s_hbm.at[idx_v.at[pl.ds(off, CHUNK)]],
                             vals_v.at[pl.ds(off, CHUNK)], sem)
        for j in range(GROUP):
            off = (g * GROUP + j) * CHUNK
            pltpu.make_async_copy(s_hbm.at[idx_v.at[pl.ds(off, CHUNK)]],
                                  vals_v.at[pl.ds(off, CHUNK)], sem).wait()
        return carry

    lax.fori_loop(0, NCHUNK // GROUP, gather_group, 0)

    zero = jnp.zeros((16,), jnp.float32)

    def accum(r, accs):
        o = r * L
        return tuple(a + vals_v[pl.ds(o + 16 * q, 16)]
                     for q, a in enumerate(accs))

    accs = lax.fori_loop(0, NB_PER_TILE, accum, (zero,) * 4)
    for q in range(4):
        acc_v[pl.ds(16 * q, 16)] = accs[q]
    pltpu.sync_copy(acc_v, out_hbm.at[wid])


def kernel(inp, emb_table, W, b):
    s = _table_dot_w(emb_table, W.reshape(-1).astype(jnp.float32))
    inp_flat = inp.reshape(-1).astype(jnp.int32)
    partials = _sc_segment_sum(inp_flat, s)
    total = partials[:, :L].sum(axis=0)
    return jax.nn.sigmoid(total * (1.0 / B) + b[0])


# speedup vs baseline: 4.4569x; 4.4569x over previous
import functools

import jax
import jax.numpy as jnp
import numpy as np
from jax import lax
from jax.experimental import pallas as pl
from jax.experimental.pallas import tpu as pltpu
from jax.experimental.pallas import tpu_sc as plsc

B = 16384
LOGB = 14
L = 50
E = 32
V = 1_000_000

BLKL = 32768
NBLK = (V + BLKL - 1) // BLKL
SPAD = NBLK * BLKL


def _dot_body(w_ref, x_ref, o_ref):
    i = pl.program_id(0)
    prod = jnp.sum(x_ref[...] * w_ref[...], axis=0)
    o_ref[pl.ds(pl.multiple_of(i * BLKL, 1024), BLKL)] = prod


def _table_dot_w(emb_table, w_row):
    xt = emb_table.T
    return pl.pallas_call(
        _dot_body,
        grid=(NBLK,),
        in_specs=[
            pl.BlockSpec((E, 1), lambda i: (0, 0)),
            pl.BlockSpec((E, BLKL), lambda i: (0, i)),
        ],
        out_specs=pl.BlockSpec((SPAD,), lambda i: (0,)),
        out_shape=jax.ShapeDtypeStruct((SPAD,), jnp.float32),
    )(w_row.reshape(E, 1), xt)


NC, NS = 2, 16
NW = NC * NS
NIDX = (B * L) // NW
CHUNK = 128
NCHUNK = NIDX // CHUNK
GROUP = 8
NRUN = 3

_mesh = plsc.VectorSubcoreMesh(core_axis_name="c", subcore_axis_name="s")


@functools.partial(
    pl.kernel,
    mesh=_mesh,
    out_type=jax.ShapeDtypeStruct((NW, NRUN * 16), jnp.float32),
    scratch_types=[
        pltpu.VMEM((NIDX,), jnp.int32),
        pltpu.VMEM((NIDX,), jnp.float32),
        pltpu.VMEM((NRUN * 16,), jnp.float32),
        pltpu.SemaphoreType.DMA,
    ],
)
def _sc_segment_sum(inp_hbm, s_hbm, out_hbm, idx_v, vals_v, acc_v, sem):
    cid = lax.axis_index("c")
    sid = lax.axis_index("s")
    wid = sid * NC + cid
    base = wid * NIDX

    pltpu.sync_copy(inp_hbm.at[pl.ds(base, NIDX)], idx_v)

    def gather_group(g, carry):
        for j in range(GROUP):
            off = (g * GROUP + j) * CHUNK
            pltpu.async_copy(s_hbm.at[idx_v.at[pl.ds(off, CHUNK)]],
                             vals_v.at[pl.ds(off, CHUNK)], sem)
        for j in range(GROUP):
            off = (g * GROUP + j) * CHUNK
            pltpu.make_async_copy(s_hbm.at[idx_v.at[pl.ds(off, CHUNK)]],
                                  vals_v.at[pl.ds(off, CHUNK)], sem).wait()
        return carry

    lax.fori_loop(0, NCHUNK // GROUP, gather_group, 0)

    seg0 = lax.shift_right_logical(base, LOGB)
    zero = jnp.zeros((16,), jnp.float32)
    vbase = base // 16

    def accum(q, accs):
        v = vals_v[pl.ds(q * 16, 16)]
        rel = lax.shift_right_logical(vbase + q, LOGB - 4) - seg0
        return tuple(
            a + v * (rel == k).astype(jnp.float32)
            for k, a in enumerate(accs))

    accs = lax.fori_loop(0, NIDX // 16, accum, (zero,) * NRUN)
    for k in range(NRUN):
        acc_v[pl.ds(16 * k, 16)] = accs[k]
    pltpu.sync_copy(acc_v, out_hbm.at[wid])


_SEG_IDS = np.minimum(
    np.array([[t * NIDX // B + k for k in range(NRUN)] for t in range(NW)],
             dtype=np.int32), L - 1)


def kernel(inp, emb_table, W, b):
    s = _table_dot_w(emb_table, W.reshape(-1).astype(jnp.float32))
    inp_flat = inp.T.reshape(-1).astype(jnp.int32)
    partials = _sc_segment_sum(inp_flat, s)
    sums = partials.reshape(NW, NRUN, 16).sum(axis=-1)
    total = jnp.zeros((L,), jnp.float32).at[_SEG_IDS.reshape(-1)].add(
        sums.reshape(-1))
    return jax.nn.sigmoid(total * (1.0 / B) + b[0])


# speedup vs baseline: 6.2178x; 1.3951x over previous
import functools

import jax
import jax.numpy as jnp
import numpy as np
from jax import lax
from jax.experimental import pallas as pl
from jax.experimental.pallas import tpu as pltpu
from jax.experimental.pallas import tpu_sc as plsc

B = 16384
LOGB = 14
L = 50
E = 32
V = 1_000_000

BLKL = 32768
NBLK = (V + BLKL - 1) // BLKL
SPAD = NBLK * BLKL


def _dot_body(w_ref, x_ref, o_ref):
    i = pl.program_id(0)
    prod = jnp.sum(x_ref[...] * w_ref[...], axis=0)
    o_ref[pl.ds(pl.multiple_of(i * BLKL, 1024), BLKL)] = prod


def _table_dot_w(emb_table, w_row):
    xt = emb_table.T
    return pl.pallas_call(
        _dot_body,
        grid=(NBLK,),
        in_specs=[
            pl.BlockSpec((E, 1), lambda i: (0, 0)),
            pl.BlockSpec((E, BLKL), lambda i: (0, i)),
        ],
        out_specs=pl.BlockSpec((SPAD,), lambda i: (0,)),
        out_shape=jax.ShapeDtypeStruct((SPAD,), jnp.float32),
    )(w_row.reshape(E, 1), xt)


NC, NS = 2, 16
NW = NC * NS
NIDX = (B * L) // NW
CHUNK = 128
NCHUNK = NIDX // CHUNK
GROUP = 8
NRUN = 3

_mesh = plsc.VectorSubcoreMesh(core_axis_name="c", subcore_axis_name="s")


@functools.partial(
    pl.kernel,
    mesh=_mesh,
    out_type=jax.ShapeDtypeStruct((NW, NRUN * 16), jnp.float32),
    scratch_types=[
        pltpu.VMEM((NIDX,), jnp.int32),
        pltpu.VMEM((NIDX,), jnp.float32),
        pltpu.VMEM((NRUN * 16,), jnp.float32),
    ] + [pltpu.SemaphoreType.DMA] * GROUP,
)
def _sc_segment_sum(inp_hbm, s_hbm, out_hbm, idx_v, vals_v, acc_v, *sems):
    cid = lax.axis_index("c")
    sid = lax.axis_index("s")
    wid = sid * NC + cid
    base = wid * NIDX

    pltpu.sync_copy(inp_hbm.at[pl.ds(base, NIDX)], idx_v)

    def fire(c, sem):
        off = c * CHUNK
        pltpu.async_copy(s_hbm.at[idx_v.at[pl.ds(off, CHUNK)]],
                         vals_v.at[pl.ds(off, CHUNK)], sem)

    def wait(c, sem):
        off = c * CHUNK
        pltpu.make_async_copy(s_hbm.at[idx_v.at[pl.ds(off, CHUNK)]],
                              vals_v.at[pl.ds(off, CHUNK)], sem).wait()

    seg0 = lax.shift_right_logical(base, LOGB)
    zero = jnp.zeros((16,), jnp.float32)
    vbase = base // 16

    def accum_chunk(c, accs):
        for qq in range(CHUNK // 16):
            q = c * (CHUNK // 16) + qq
            v = vals_v[pl.ds(q * 16, 16)]
            rel = lax.shift_right_logical(vbase + q, LOGB - 4) - seg0
            accs = tuple(a + v * (rel == k).astype(jnp.float32)
                         for k, a in enumerate(accs))
        return accs

    for j in range(GROUP):
        fire(j, sems[j])

    def body(g, accs):
        for j in range(GROUP):
            c = g * GROUP + j
            wait(c, sems[j])
            accs = accum_chunk(c, accs)
            fire(c + GROUP, sems[j])
        return accs

    accs = lax.fori_loop(0, NCHUNK // GROUP - 1, body, (zero,) * NRUN)
    for j in range(GROUP):
        c = NCHUNK - GROUP + j
        wait(c, sems[j])
        accs = accum_chunk(c, accs)

    for k in range(NRUN):
        acc_v[pl.ds(16 * k, 16)] = accs[k]
    pltpu.sync_copy(acc_v, out_hbm.at[wid])


_SEG_MAT = np.zeros((NW * NRUN, L), dtype=np.float32)
for _t in range(NW):
    for _k in range(NRUN):
        _l = _t * NIDX // B + _k
        if _l < L:
            _SEG_MAT[_t * NRUN + _k, _l] = 1.0


def kernel(inp, emb_table, W, b):
    s = _table_dot_w(emb_table, W.reshape(-1).astype(jnp.float32))
    inp_flat = inp.T.reshape(-1).astype(jnp.int32)
    partials = _sc_segment_sum(inp_flat, s)
    sums = partials.reshape(NW, NRUN, 16).sum(axis=-1)
    total = sums.reshape(1, NW * NRUN) @ jnp.asarray(_SEG_MAT)
    return jax.nn.sigmoid(total.reshape(L) * (1.0 / B) + b[0])


# speedup vs baseline: 6.7403x; 1.0840x over previous
import functools

import jax
import jax.numpy as jnp
import numpy as np
from jax import lax
from jax.experimental import pallas as pl
from jax.experimental.pallas import tpu as pltpu
from jax.experimental.pallas import tpu_sc as plsc

B = 16384
LOGB = 14
L = 50
E = 32
V = 1_000_000

BLKL = 65536
NBLK = (V + BLKL - 1) // BLKL
SPAD = NBLK * BLKL


def _dot_body(w_ref, x_ref, o_ref):
    i = pl.program_id(0)
    prod = jnp.sum(x_ref[...] * w_ref[...], axis=0)
    o_ref[pl.ds(pl.multiple_of(i * BLKL, 1024), BLKL)] = prod


def _table_dot_w(emb_table, w_row):
    xt = emb_table.T
    return pl.pallas_call(
        _dot_body,
        grid=(NBLK,),
        in_specs=[
            pl.BlockSpec((E, 1), lambda i: (0, 0)),
            pl.BlockSpec((E, BLKL), lambda i: (0, i)),
        ],
        out_specs=pl.BlockSpec((SPAD,), lambda i: (0,)),
        out_shape=jax.ShapeDtypeStruct((SPAD,), jnp.float32),
    )(w_row.reshape(E, 1), xt)


NC, NS = 2, 16
NW = NC * NS
NIDX = (B * L) // NW
CHUNK = 128
NCHUNK = NIDX // CHUNK
GROUP = 20
NRUN = 3
assert NCHUNK % GROUP == 0, "rolling window must tile the chunk count"

_mesh = plsc.VectorSubcoreMesh(core_axis_name="c", subcore_axis_name="s")


@functools.partial(
    pl.kernel,
    mesh=_mesh,
    out_type=jax.ShapeDtypeStruct((NW, NRUN * 16), jnp.float32),
    scratch_types=[
        pltpu.VMEM((NIDX,), jnp.int32),
        pltpu.VMEM((NIDX,), jnp.float32),
        pltpu.VMEM((NRUN * 16,), jnp.float32),
    ] + [pltpu.SemaphoreType.DMA] * GROUP,
)
def _sc_segment_sum(inp_hbm, s_hbm, out_hbm, idx_v, vals_v, acc_v, *sems):
    cid = lax.axis_index("c")
    sid = lax.axis_index("s")
    wid = sid * NC + cid
    base = wid * NIDX

    pltpu.sync_copy(inp_hbm.at[pl.ds(base, NIDX)], idx_v)

    def fire(c, sem):
        off = c * CHUNK
        pltpu.async_copy(s_hbm.at[idx_v.at[pl.ds(off, CHUNK)]],
                         vals_v.at[pl.ds(off, CHUNK)], sem)

    def wait(c, sem):
        off = c * CHUNK
        pltpu.make_async_copy(s_hbm.at[idx_v.at[pl.ds(off, CHUNK)]],
                              vals_v.at[pl.ds(off, CHUNK)], sem).wait()

    seg0 = lax.shift_right_logical(base, LOGB)
    zero = jnp.zeros((16,), jnp.float32)
    vbase = base // 16

    def accum_chunk(c, accs):
        for qq in range(CHUNK // 16):
            q = c * (CHUNK // 16) + qq
            v = vals_v[pl.ds(q * 16, 16)]
            rel = lax.shift_right_logical(vbase + q, LOGB - 4) - seg0
            accs = tuple(a + v * (rel == k).astype(jnp.float32)
                         for k, a in enumerate(accs))
        return accs

    for j in range(GROUP):
        fire(j, sems[j])

    def body(g, accs):
        for j in range(GROUP):
            c = g * GROUP + j
            wait(c, sems[j])
            accs = accum_chunk(c, accs)
            fire(c + GROUP, sems[j])
        return accs

    accs = lax.fori_loop(0, NCHUNK // GROUP - 1, body, (zero,) * NRUN)
    for j in range(GROUP):
        c = NCHUNK - GROUP + j
        wait(c, sems[j])
        accs = accum_chunk(c, accs)

    for k in range(NRUN):
        acc_v[pl.ds(16 * k, 16)] = accs[k]
    pltpu.sync_copy(acc_v, out_hbm.at[wid])


_SEG_MAT = np.zeros((NW * NRUN, L), dtype=np.float32)
for _t in range(NW):
    for _k in range(NRUN):
        _l = _t * NIDX // B + _k
        if _l < L:
            _SEG_MAT[_t * NRUN + _k, _l] = 1.0


def kernel(inp, emb_table, W, b):
    s = _table_dot_w(emb_table, W.reshape(-1).astype(jnp.float32))
    inp_flat = inp.T.reshape(-1).astype(jnp.int32)
    partials = _sc_segment_sum(inp_flat, s)
    sums = partials.reshape(NW, NRUN, 16).sum(axis=-1)
    total = sums.reshape(1, NW * NRUN) @ jnp.asarray(_SEG_MAT)
    return jax.nn.sigmoid(total.reshape(L) * (1.0 / B) + b[0])
